# baseline (speedup 1.0000x reference)
import functools
import math

import jax
import jax.numpy as jnp
from jax import lax
from jax.experimental import pallas as pl
from jax.experimental.pallas import tpu as pltpu
from jax.experimental.pallas import tpu_sc as plsc

D_MODEL = 1024
SCALE = math.sqrt(D_MODEL)
L = 16
NBUF = 14
LEAD = 7


@functools.cache
def _make_kernel(B: int, D: int):
    NC, NS = 2, 16
    NW = NC * NS
    assert B % NW == 0
    b_per_w = B // NW
    C = 8
    n_chunks = b_per_w // C
    mesh = plsc.VectorSubcoreMesh(core_axis_name="c", subcore_axis_name="s")

    @functools.partial(
        pl.kernel,
        out_type=jax.ShapeDtypeStruct((B, D), jnp.float32),
        mesh=mesh,
        scratch_types=[
            pltpu.VMEM((b_per_w,), jnp.int32),
            pltpu.VMEM((NBUF, C, D), jnp.float32),
            pltpu.SemaphoreType.DMA((NBUF,)),
            pltpu.SemaphoreType.DMA((NBUF,)),
        ],
    )
    def emb_kernel(table_hbm, idx_hbm, out_hbm, idx_v, bufs, in_sems, out_sems):
        wid = lax.axis_index("s") * NC + lax.axis_index("c")
        base = wid * b_per_w
        pltpu.sync_copy(idx_hbm.at[pl.ds(base, b_per_w)], idx_v)

        def start_in(g, b):
            pltpu.async_copy(
                table_hbm.at[idx_v.at[pl.ds(g * C, C)]], bufs.at[b], in_sems.at[b]
            )

        def wait_in(b):
            pltpu.make_async_copy(
                table_hbm.at[pl.ds(0, C)], bufs.at[b], in_sems.at[b]
            ).wait()

        def start_out(g, b):
            pltpu.async_copy(
                bufs.at[b], out_hbm.at[pl.ds(base + g * C, C)], out_sems.at[b]
            )

        def wait_out(b):
            pltpu.make_async_copy(
                bufs.at[b], out_hbm.at[pl.ds(base, C)], out_sems.at[b]
            ).wait()

        for g in range(LEAD):
            start_in(g, g % NBUF)

        def body(g, _):
            b = lax.rem(g, NBUF)
            ga = g + LEAD
            ba = lax.rem(ga, NBUF)

            @pl.when(ga < n_chunks)
            def _():
                @pl.when(ga >= NBUF)
                def _():
                    wait_out(ba)
                start_in(ga, ba)

            wait_in(b)

            def scale_row(r, _):
                for j in range(D // L):
                    sl = pl.ds(j * L, L)
                    bufs[b, r, sl] = bufs[b, r, sl] * SCALE
                return 0

            lax.fori_loop(0, C, scale_row, 0)
            start_out(g, b)
            return 0

        lax.fori_loop(0, n_chunks, body, 0)

        for g in range(n_chunks - NBUF, n_chunks):
            wait_out(g % NBUF)

    return emb_kernel


def kernel(x, table):
    B = x.shape[0] * x.shape[1]
    D = table.shape[1]
    idx = x.reshape(B).astype(jnp.int32)
    out = _make_kernel(B, D)(table, idx)
    return out.reshape(x.shape[0], x.shape[1], D)


# speedup vs baseline: 2.5391x; 2.5391x over previous
import functools
import math

import jax
import jax.numpy as jnp
from jax import lax
from jax.experimental import pallas as pl
from jax.experimental.pallas import tpu as pltpu
from jax.experimental.pallas import tpu_sc as plsc

D_MODEL = 1024
SCALE = math.sqrt(D_MODEL)
L = 16
NBUF = 7
LEAD = 4


@functools.cache
def _make_kernel(B: int, D: int):
    NC, NS = 2, 16
    NW = NC * NS
    assert B % NW == 0
    b_per_w = B // NW
    C = 16
    n_chunks = b_per_w // C
    mesh = plsc.VectorSubcoreMesh(core_axis_name="c", subcore_axis_name="s")

    @functools.partial(
        pl.kernel,
        out_type=jax.ShapeDtypeStruct((B, D), jnp.float32),
        mesh=mesh,
        scratch_types=[
            pltpu.VMEM((b_per_w,), jnp.int32),
            pltpu.VMEM((NBUF, C, D), jnp.float32),
            pltpu.SemaphoreType.DMA((NBUF,)),
            pltpu.SemaphoreType.DMA((NBUF,)),
        ],
    )
    def emb_kernel(table_hbm, idx_hbm, out_hbm, idx_v, bufs, in_sems, out_sems):
        wid = lax.axis_index("s") * NC + lax.axis_index("c")
        base = wid * b_per_w
        pltpu.sync_copy(idx_hbm.at[pl.ds(base, b_per_w)], idx_v)

        def start_in(g, b):
            pltpu.async_copy(
                table_hbm.at[idx_v.at[pl.ds(g * C, C)]], bufs.at[b], in_sems.at[b]
            )

        def wait_in(b):
            pltpu.make_async_copy(
                table_hbm.at[pl.ds(0, C)], bufs.at[b], in_sems.at[b]
            ).wait()

        def start_out(g, b):
            pltpu.async_copy(
                bufs.at[b], out_hbm.at[pl.ds(base + g * C, C)], out_sems.at[b]
            )

        def wait_out(b):
            pltpu.make_async_copy(
                bufs.at[b], out_hbm.at[pl.ds(base, C)], out_sems.at[b]
            ).wait()

        for g in range(LEAD):
            start_in(g, g % NBUF)

        def body(g, _):
            b = lax.rem(g, NBUF)
            ga = g + LEAD
            ba = lax.rem(ga, NBUF)

            @pl.when(ga < n_chunks)
            def _():
                @pl.when(ga >= NBUF)
                def _():
                    wait_out(ba)
                start_in(ga, ba)

            wait_in(b)

            def scale_row(r, _):
                for j in range(D // L):
                    sl = pl.ds(j * L, L)
                    bufs[b, r, sl] = bufs[b, r, sl] * SCALE
                return 0

            lax.fori_loop(0, C, scale_row, 0)
            start_out(g, b)
            return 0

        lax.fori_loop(0, n_chunks, body, 0)

        for g in range(n_chunks - NBUF, n_chunks):
            wait_out(g % NBUF)

    return emb_kernel


def kernel(x, table):
    B = x.shape[0] * x.shape[1]
    D = table.shape[1]
    idx = x.reshape(B).astype(jnp.int32)
    out = _make_kernel(B, D)(table, idx)
    return out.reshape(x.shape[0], x.shape[1], D)
